# baseline (speedup 1.0000x reference)
import functools

import jax
import jax.numpy as jnp
from jax import lax
from jax.experimental import pallas as pl
from jax.experimental.pallas import tpu as pltpu
from jax.experimental.pallas import tpu_sc as plsc

N_NODES = 10000
D = 128
DH = D // 2
N_EDGES = 320000

NC = 2
NS = 16
L = 16
NW = NC * NS
EPW = N_EDGES // NW
CHUNK = 100
NCHUNK = EPW // CHUNK
N_PAD = 10000
ROWS_PT = N_PAD // NS


def _mm_body(x_ref, w_ref, o_ref):
    o_ref[0] = jnp.dot(x_ref[...], w_ref[0],
                       preferred_element_type=jnp.float32)


def _matmul_split(x, W_split):
    blk = N_NODES // 10
    return pl.pallas_call(
        _mm_body,
        grid=(10, 2),
        in_specs=[pl.BlockSpec((blk, D), lambda i, f: (i, 0)),
                  pl.BlockSpec((1, D, DH), lambda i, f: (f, 0, 0))],
        out_specs=pl.BlockSpec((1, blk, DH), lambda i, f: (f, i, 0)),
        out_shape=jax.ShapeDtypeStruct((2, N_NODES, DH), jnp.float32),
    )(x, W_split)


def _add_body(a0_ref, a1_ref, b0_ref, b1_ref, o_ref):
    o_ref[:, :DH] = a0_ref[0] + b0_ref[0]
    o_ref[:, DH:] = a1_ref[0] + b1_ref[0]


def _combine(parts):
    blk = N_NODES // 10
    spec = lambda f: pl.BlockSpec((1, blk, DH), lambda i, f=f: (f, i, 0))
    return pl.pallas_call(
        _add_body,
        grid=(10,),
        in_specs=[spec(0), spec(1), spec(0), spec(1)],
        out_specs=pl.BlockSpec((blk, D), lambda i: (i, 0)),
        out_shape=jax.ShapeDtypeStruct((N_NODES, D), jnp.float32),
    )(parts[0], parts[0], parts[1], parts[1])


def _edge_agg(xw_split, src2, dst2, w2):
    mesh = plsc.VectorSubcoreMesh(core_axis_name="c", subcore_axis_name="s",
                                  num_cores=NC, num_subcores=NS)
    CPW = NCHUNK

    @functools.partial(
        pl.kernel,
        out_type=jax.ShapeDtypeStruct((NC, 2, N_PAD, DH), jnp.float32),
        mesh=mesh,
        compiler_params=pltpu.CompilerParams(use_tc_tiling_on_sc=False),
        scratch_types=[
            pltpu.VMEM((CPW, CHUNK), jnp.int32),
            pltpu.VMEM((CPW, CHUNK), jnp.int32),
            pltpu.VMEM((CPW, CHUNK), jnp.float32),
            pltpu.VMEM((CHUNK, DH), jnp.float32),
            pltpu.VMEM((CHUNK, DH), jnp.float32),
            pltpu.VMEM((CHUNK, DH), jnp.float32),
            pltpu.VMEM((ROWS_PT, DH), jnp.float32),
            pltpu.VMEM_SHARED((N_PAD, DH), jnp.float32),
            pltpu.SemaphoreType.DMA,
            pltpu.SemaphoreType.DMA,
        ],
    )
    def k(xw_hbm, src_hbm, dst_hbm, w_hbm, out_hbm,
          src_v, dst_v, w_v, rows_a, rows_b, scaled_a, zero_v, acc,
          sem_a, sem_b):
        scaled_b = scaled_a
        ssem_a = ssem_b = None
        c = lax.axis_index("c")
        s = lax.axis_index("s")
        wid = c * NS + s

        pltpu.sync_copy(src_hbm.at[pl.ds(wid * CPW, CPW)], src_v)
        pltpu.sync_copy(dst_hbm.at[pl.ds(wid * CPW, CPW)], dst_v)
        pltpu.sync_copy(w_hbm.at[pl.ds(wid * CPW, CPW)], w_v)

        @pl.loop(0, ROWS_PT)
        def _zero(r):
            for g in range(DH // L):
                zero_v[r, pl.ds(g * L, L)] = jnp.zeros((L,), jnp.float32)

        for f in range(2):
            xw_f = xw_hbm.at[f]

            def start_gather(kk, buf, sem):
                pltpu.async_copy(xw_f.at[src_v.at[kk]], buf, sem)

            def wait_scatter(kk, scaled, ssem):
                pltpu.make_async_copy(scaled, acc.at[dst_v.at[kk]], ssem).wait()

            def finish_chunk(kk, buf, sem, scaled, ssem):
                pltpu.make_async_copy(xw_f.at[src_v.at[kk]], buf, sem).wait()

                def scale_group(base):
                    w16 = w_v[kk, pl.ds(base, L)]
                    for e in range(L):
                        wt = w16[e]
                        row = base + e
                        for q in range(DH // L):
                            sl = pl.ds(q * L, L)
                            scaled[row, sl] = buf[row, sl] * wt

                @pl.loop(0, CHUNK // L)
                def _scale(g):
                    scale_group(g * L)

                if CHUNK % L:
                    scale_group(CHUNK - L)

                pltpu.sync_copy(scaled, acc.at[dst_v.at[kk]], add=True)

            pltpu.sync_copy(zero_v, acc.at[pl.ds(s * ROWS_PT, ROWS_PT)])
            plsc.subcore_barrier()

            start_gather(0, rows_a, sem_a)

            @pl.loop(0, NCHUNK // 2)
            def _pair(p):
                k0 = 2 * p
                start_gather(k0 + 1, rows_b, sem_b)
                finish_chunk(k0, rows_a, sem_a, scaled_a, ssem_a)

                @pl.when(p < NCHUNK // 2 - 1)
                def _prefetch():
                    start_gather(k0 + 2, rows_a, sem_a)

                finish_chunk(k0 + 1, rows_b, sem_b, scaled_b, ssem_b)

            plsc.subcore_barrier()
            pltpu.sync_copy(acc.at[pl.ds(s * ROWS_PT, ROWS_PT)],
                            out_hbm.at[c].at[f].at[pl.ds(s * ROWS_PT, ROWS_PT)])

    return k(xw_split, src2, dst2, w2)


def kernel(x, edge_index, edge_weight, W):
    rows = N_EDGES // CHUNK
    dst2 = edge_index[0].astype(jnp.int32).reshape(rows, CHUNK)
    src2 = edge_index[1].astype(jnp.int32).reshape(rows, CHUNK)
    w2 = edge_weight.reshape(rows, CHUNK)
    W_split = jnp.stack([W[:, :DH], W[:, DH:]], axis=0)
    xw_split = _matmul_split(x, W_split)
    parts = _edge_agg(xw_split, src2, dst2, w2)
    return _combine(parts)


# speedup vs baseline: 1.1079x; 1.1079x over previous
import functools

import jax
import jax.numpy as jnp
from jax import lax
from jax.experimental import pallas as pl
from jax.experimental.pallas import tpu as pltpu
from jax.experimental.pallas import tpu_sc as plsc

N_NODES = 10000
D = 128
DH = D // 2
N_EDGES = 320000

NC = 2
NS = 16
L = 16
NW = NC * NS
EPW = N_EDGES // NW
CHUNK = 100
NCHUNK = EPW // CHUNK
N_PAD = 10000
ROWS_PT = N_PAD // NS


def _mm_body(x_ref, w_ref, o_ref):
    o_ref[0] = jnp.dot(x_ref[...], w_ref[0],
                       preferred_element_type=jnp.float32)


def _matmul_split(x, W_split):
    blk = N_NODES // 10
    return pl.pallas_call(
        _mm_body,
        grid=(10, 2),
        in_specs=[pl.BlockSpec((blk, D), lambda i, f: (i, 0)),
                  pl.BlockSpec((1, D, DH), lambda i, f: (f, 0, 0))],
        out_specs=pl.BlockSpec((1, blk, DH), lambda i, f: (f, i, 0)),
        out_shape=jax.ShapeDtypeStruct((2, N_NODES, DH), jnp.float32),
    )(x, W_split)


def _add_body(a0_ref, a1_ref, b0_ref, b1_ref, o_ref):
    o_ref[:, :DH] = a0_ref[0] + b0_ref[0]
    o_ref[:, DH:] = a1_ref[0] + b1_ref[0]


def _combine(parts):
    blk = N_NODES // 10
    spec = lambda f: pl.BlockSpec((1, blk, DH), lambda i, f=f: (f, i, 0))
    return pl.pallas_call(
        _add_body,
        grid=(10,),
        in_specs=[spec(0), spec(1), spec(0), spec(1)],
        out_specs=pl.BlockSpec((blk, D), lambda i: (i, 0)),
        out_shape=jax.ShapeDtypeStruct((N_NODES, D), jnp.float32),
    )(parts[0], parts[0], parts[1], parts[1])


def _edge_agg(xw_split, src2, dst2, w2):
    mesh = plsc.VectorSubcoreMesh(core_axis_name="c", subcore_axis_name="s",
                                  num_cores=NC, num_subcores=NS)
    CPW = NCHUNK

    @functools.partial(
        pl.kernel,
        out_type=jax.ShapeDtypeStruct((NC, 2, N_PAD, DH), jnp.float32),
        mesh=mesh,
        compiler_params=pltpu.CompilerParams(use_tc_tiling_on_sc=False),
        scratch_types=[
            pltpu.VMEM((CPW, CHUNK), jnp.int32),
            pltpu.VMEM((CPW, CHUNK), jnp.int32),
            pltpu.VMEM((CPW, CHUNK), jnp.float32),
            pltpu.VMEM((CHUNK, DH), jnp.float32),
            pltpu.VMEM((CHUNK, DH), jnp.float32),
            pltpu.VMEM((CHUNK, DH), jnp.float32),
            pltpu.VMEM((CHUNK, DH), jnp.float32),
            pltpu.VMEM((CHUNK, DH), jnp.float32),
            pltpu.VMEM((ROWS_PT // 5, DH), jnp.float32),
            pltpu.VMEM_SHARED((N_PAD, DH), jnp.float32),
            pltpu.SemaphoreType.DMA,
            pltpu.SemaphoreType.DMA,
            pltpu.SemaphoreType.DMA,
            pltpu.SemaphoreType.DMA,
        ],
    )
    def k(xw_hbm, src_hbm, dst_hbm, w_hbm, out_hbm,
          src_v, dst_v, w_v, rows_a, rows_b, rows_c, rows_d, scaled_v, zero_v,
          acc, sem_a, sem_b, sem_c, sem_d):
        c = lax.axis_index("c")
        s = lax.axis_index("s")
        wid = c * NS + s

        pltpu.sync_copy(src_hbm.at[pl.ds(wid * CPW, CPW)], src_v)
        pltpu.sync_copy(dst_hbm.at[pl.ds(wid * CPW, CPW)], dst_v)
        pltpu.sync_copy(w_hbm.at[pl.ds(wid * CPW, CPW)], w_v)

        @pl.loop(0, ROWS_PT // 5)
        def _zero(r):
            for g in range(DH // L):
                zero_v[r, pl.ds(g * L, L)] = jnp.zeros((L,), jnp.float32)

        for f in range(2):
            xw_f = xw_hbm.at[f]

            def start_gather(kk, buf, sem):
                pltpu.async_copy(xw_f.at[src_v.at[kk]], buf, sem)

            def finish_chunk(kk, buf, sem):
                pltpu.make_async_copy(xw_f.at[src_v.at[kk]], buf, sem).wait()

                def scale_group(base):
                    w16 = w_v[kk, pl.ds(base, L)]
                    for e in range(L):
                        wt = w16[e]
                        row = base + e
                        for q in range(DH // L):
                            sl = pl.ds(q * L, L)
                            scaled_v[row, sl] = buf[row, sl] * wt

                @pl.loop(0, CHUNK // L)
                def _scale(g):
                    scale_group(g * L)

                if CHUNK % L:
                    scale_group(CHUNK - L)

                pltpu.sync_copy(scaled_v, acc.at[dst_v.at[kk]], add=True)

            for z in range(5):
                pltpu.sync_copy(
                    zero_v,
                    acc.at[pl.ds(s * ROWS_PT + z * (ROWS_PT // 5),
                                 ROWS_PT // 5)])
            plsc.subcore_barrier()

            start_gather(0, rows_a, sem_a)
            start_gather(1, rows_b, sem_b)
            start_gather(2, rows_c, sem_c)

            bufs = ((rows_a, sem_a), (rows_b, sem_b),
                    (rows_c, sem_c), (rows_d, sem_d))

            @pl.loop(0, NCHUNK // 4)
            def _quad(q):
                k0 = 4 * q
                for j in range(4):
                    nxt = k0 + j + 3
                    buf, sem = bufs[(j + 3) % 4]
                    if j == 0:
                        start_gather(nxt, buf, sem)
                    else:
                        @pl.when(nxt < NCHUNK)
                        def _prefetch(nxt=nxt, buf=buf, sem=sem):
                            start_gather(nxt, buf, sem)
                    finish_chunk(k0 + j, *bufs[j])

            plsc.subcore_barrier()
            pltpu.sync_copy(acc.at[pl.ds(s * ROWS_PT, ROWS_PT)],
                            out_hbm.at[c].at[f].at[pl.ds(s * ROWS_PT, ROWS_PT)])

    return k(xw_split, src2, dst2, w2)


def kernel(x, edge_index, edge_weight, W):
    rows = N_EDGES // CHUNK
    dst2 = edge_index[0].astype(jnp.int32).reshape(rows, CHUNK)
    src2 = edge_index[1].astype(jnp.int32).reshape(rows, CHUNK)
    w2 = edge_weight.reshape(rows, CHUNK)
    W_split = jnp.stack([W[:, :DH], W[:, DH:]], axis=0)
    xw_split = _matmul_split(x, W_split)
    parts = _edge_agg(xw_split, src2, dst2, w2)
    return _combine(parts)


# speedup vs baseline: 1.7140x; 1.5470x over previous
import functools

import jax
import jax.numpy as jnp
from jax import lax
from jax.experimental import pallas as pl
from jax.experimental.pallas import tpu as pltpu
from jax.experimental.pallas import tpu_sc as plsc

N_NODES = 10000
D = 128
N_EDGES = 320000

NC = 2
NS = 16
L = 16
NW = NC * NS
EPW = N_EDGES // NW
CHUNK = 100
NCHUNK = EPW // CHUNK
ROWS_PT = N_NODES // NS
ZROWS = ROWS_PT // 5


def _mm_body(x_ref, w_ref, o_ref):
    o_ref[...] = jnp.dot(x_ref[...], w_ref[...],
                         preferred_element_type=jnp.float32
                         ).astype(jnp.bfloat16)


def _matmul_bf16(x, W):
    blk = N_NODES // 5
    return pl.pallas_call(
        _mm_body,
        grid=(5,),
        in_specs=[pl.BlockSpec((blk, D), lambda i: (i, 0)),
                  pl.BlockSpec((D, D), lambda i: (0, 0))],
        out_specs=pl.BlockSpec((blk, D), lambda i: (i, 0)),
        out_shape=jax.ShapeDtypeStruct((N_NODES, D), jnp.bfloat16),
    )(x, W)


def _add_body(a_ref, b_ref, o_ref):
    o_ref[...] = (a_ref[...].astype(jnp.float32)
                  + b_ref[...].astype(jnp.float32))


def _combine(parts):
    blk = N_NODES // 5
    return pl.pallas_call(
        _add_body,
        grid=(5,),
        in_specs=[pl.BlockSpec((blk, D), lambda i: (i, 0)),
                  pl.BlockSpec((blk, D), lambda i: (i, 0))],
        out_specs=pl.BlockSpec((blk, D), lambda i: (i, 0)),
        out_shape=jax.ShapeDtypeStruct((N_NODES, D), jnp.float32),
    )(parts[0], parts[1])


def _edge_agg(xw, src2, dst2, w2):
    mesh = plsc.VectorSubcoreMesh(core_axis_name="c", subcore_axis_name="s",
                                  num_cores=NC, num_subcores=NS)
    CPW = NCHUNK

    @functools.partial(
        pl.kernel,
        out_type=jax.ShapeDtypeStruct((NC, N_NODES, D), jnp.bfloat16),
        mesh=mesh,
        compiler_params=pltpu.CompilerParams(use_tc_tiling_on_sc=False,
                                             needs_layout_passes=False),
        scratch_types=[
            pltpu.VMEM((CPW, CHUNK), jnp.int32),
            pltpu.VMEM((CPW, CHUNK), jnp.int32),
            pltpu.VMEM((CPW, CHUNK), jnp.float32),
            pltpu.VMEM((CHUNK, D), jnp.bfloat16),
            pltpu.VMEM((CHUNK, D), jnp.bfloat16),
            pltpu.VMEM((CHUNK, D), jnp.bfloat16),
            pltpu.VMEM((CHUNK, D), jnp.bfloat16),
            pltpu.VMEM((CHUNK, D), jnp.bfloat16),
            pltpu.VMEM((ZROWS, D), jnp.bfloat16),
            pltpu.VMEM_SHARED((N_NODES, D), jnp.bfloat16),
            pltpu.SemaphoreType.DMA,
            pltpu.SemaphoreType.DMA,
            pltpu.SemaphoreType.DMA,
            pltpu.SemaphoreType.DMA,
        ],
    )
    def k(xw_hbm, src_hbm, dst_hbm, w_hbm, out_hbm,
          src_v, dst_v, w_v, rows_a, rows_b, rows_c, rows_d, scaled_v, zero_v,
          acc, sem_a, sem_b, sem_c, sem_d):
        c = lax.axis_index("c")
        s = lax.axis_index("s")
        wid = c * NS + s

        pltpu.sync_copy(src_hbm.at[pl.ds(wid * CPW, CPW)], src_v)
        pltpu.sync_copy(dst_hbm.at[pl.ds(wid * CPW, CPW)], dst_v)
        pltpu.sync_copy(w_hbm.at[pl.ds(wid * CPW, CPW)], w_v)

        @pl.loop(0, ZROWS)
        def _zero(r):
            for g in range(D // (2 * L)):
                zero_v[r, pl.ds(g * 2 * L, 2 * L)] = jnp.zeros(
                    (2 * L,), jnp.bfloat16)

        for z in range(ROWS_PT // ZROWS):
            pltpu.sync_copy(zero_v,
                            acc.at[pl.ds(s * ROWS_PT + z * ZROWS, ZROWS)])
        plsc.subcore_barrier()

        def start_gather(kk, buf, sem):
            pltpu.async_copy(xw_hbm.at[src_v.at[kk]], buf, sem)

        def finish_chunk(kk, buf, sem):
            pltpu.make_async_copy(xw_hbm.at[src_v.at[kk]], buf, sem).wait()

            def scale_group(base):
                w16 = w_v[kk, pl.ds(base, L)]
                for e in range(L):
                    wsp = jnp.full((L,), w16[e], jnp.float32)
                    wb = plsc.pack(wsp, wsp,
                                   format=plsc.PackFormat.INTERLEAVED)
                    row = base + e
                    for q in range(D // (2 * L)):
                        sl = pl.ds(q * 2 * L, 2 * L)
                        scaled_v[row, sl] = buf[row, sl] * wb

            @pl.loop(0, CHUNK // L)
            def _scale(g):
                scale_group(g * L)

            if CHUNK % L:
                scale_group(CHUNK - L)

            pltpu.sync_copy(scaled_v, acc.at[dst_v.at[kk]], add=True)

        start_gather(0, rows_a, sem_a)
        start_gather(1, rows_b, sem_b)
        start_gather(2, rows_c, sem_c)

        bufs = ((rows_a, sem_a), (rows_b, sem_b),
                (rows_c, sem_c), (rows_d, sem_d))

        @pl.loop(0, NCHUNK // 4)
        def _quad(q):
            k0 = 4 * q
            for j in range(4):
                nxt = k0 + j + 3
                buf, sem = bufs[(j + 3) % 4]
                if j == 0:
                    start_gather(nxt, buf, sem)
                else:
                    @pl.when(nxt < NCHUNK)
                    def _prefetch(nxt=nxt, buf=buf, sem=sem):
                        start_gather(nxt, buf, sem)
                finish_chunk(k0 + j, *bufs[j])

        plsc.subcore_barrier()
        pltpu.sync_copy(acc.at[pl.ds(s * ROWS_PT, ROWS_PT)],
                        out_hbm.at[c].at[pl.ds(s * ROWS_PT, ROWS_PT)])

    return k(xw, src2, dst2, w2)


def kernel(x, edge_index, edge_weight, W):
    rows = N_EDGES // CHUNK
    dst2 = edge_index[0].astype(jnp.int32).reshape(rows, CHUNK)
    src2 = edge_index[1].astype(jnp.int32).reshape(rows, CHUNK)
    w2 = edge_weight.reshape(rows, CHUNK)
    xw = _matmul_bf16(x, W)
    parts = _edge_agg(xw, src2, dst2, w2)
    return _combine(parts)


# speedup vs baseline: 1.8217x; 1.0629x over previous
import functools

import jax
import jax.numpy as jnp
from jax import lax
from jax.experimental import pallas as pl
from jax.experimental.pallas import tpu as pltpu
from jax.experimental.pallas import tpu_sc as plsc

N_NODES = 10000
D = 128
N_EDGES = 320000

NC = 2
NS = 16
L = 16
NW = NC * NS
EPW = N_EDGES // NW
CHUNK = 100
NCHUNK = EPW // CHUNK
ROWS_PT = N_NODES // NS
ZROWS = ROWS_PT // 5


def _mm_body(x_ref, w_ref, o_ref):
    o_ref[...] = jnp.dot(x_ref[...], w_ref[...],
                         preferred_element_type=jnp.float32
                         ).astype(jnp.bfloat16)


def _matmul_bf16(x, W):
    blk = N_NODES // 5
    return pl.pallas_call(
        _mm_body,
        grid=(5,),
        in_specs=[pl.BlockSpec((blk, D), lambda i: (i, 0)),
                  pl.BlockSpec((D, D), lambda i: (0, 0))],
        out_specs=pl.BlockSpec((blk, D), lambda i: (i, 0)),
        out_shape=jax.ShapeDtypeStruct((N_NODES, D), jnp.bfloat16),
    )(x, W)


def _add_body(a_ref, b_ref, o_ref):
    o_ref[...] = (a_ref[...].astype(jnp.float32)
                  + b_ref[...].astype(jnp.float32))


def _combine(parts):
    blk = N_NODES // 5
    return pl.pallas_call(
        _add_body,
        grid=(5,),
        in_specs=[pl.BlockSpec((blk, D), lambda i: (i, 0)),
                  pl.BlockSpec((blk, D), lambda i: (i, 0))],
        out_specs=pl.BlockSpec((blk, D), lambda i: (i, 0)),
        out_shape=jax.ShapeDtypeStruct((N_NODES, D), jnp.float32),
    )(parts[0], parts[1])


def _edge_agg(xw, src2, dst2, w2):
    mesh = plsc.VectorSubcoreMesh(core_axis_name="c", subcore_axis_name="s",
                                  num_cores=NC, num_subcores=NS)
    CPW = NCHUNK

    @functools.partial(
        pl.kernel,
        out_type=jax.ShapeDtypeStruct((NC, N_NODES, D), jnp.bfloat16),
        mesh=mesh,
        compiler_params=pltpu.CompilerParams(use_tc_tiling_on_sc=False,
                                             needs_layout_passes=False),
        scratch_types=[
            pltpu.VMEM((CPW, CHUNK), jnp.int32),
            pltpu.VMEM((CPW, CHUNK), jnp.int32),
            pltpu.VMEM((CPW, CHUNK), jnp.float32),
            pltpu.VMEM((CHUNK, D), jnp.bfloat16),
            pltpu.VMEM((CHUNK, D), jnp.bfloat16),
            pltpu.VMEM((CHUNK, D), jnp.bfloat16),
            pltpu.VMEM((CHUNK, D), jnp.bfloat16),
            pltpu.VMEM((CHUNK, D), jnp.bfloat16),
            pltpu.VMEM((CHUNK, D), jnp.bfloat16),
            pltpu.VMEM((ZROWS, D), jnp.bfloat16),
            pltpu.VMEM_SHARED((N_NODES, D), jnp.bfloat16),
            pltpu.SemaphoreType.DMA,
            pltpu.SemaphoreType.DMA,
            pltpu.SemaphoreType.DMA,
            pltpu.SemaphoreType.DMA,
            pltpu.SemaphoreType.DMA,
            pltpu.SemaphoreType.DMA,
        ],
    )
    def k(xw_hbm, src_hbm, dst_hbm, w_hbm, out_hbm,
          src_v, dst_v, w_v, rows_a, rows_b, rows_c, rows_d, scaled_a,
          scaled_b, zero_v, acc, sem_a, sem_b, sem_c, sem_d, ssem_a, ssem_b):
        c = lax.axis_index("c")
        s = lax.axis_index("s")
        wid = c * NS + s

        pltpu.sync_copy(src_hbm.at[pl.ds(wid * CPW, CPW)], src_v)
        pltpu.sync_copy(dst_hbm.at[pl.ds(wid * CPW, CPW)], dst_v)
        pltpu.sync_copy(w_hbm.at[pl.ds(wid * CPW, CPW)], w_v)

        @pl.loop(0, ZROWS)
        def _zero(r):
            for g in range(D // (2 * L)):
                zero_v[r, pl.ds(g * 2 * L, 2 * L)] = jnp.zeros(
                    (2 * L,), jnp.bfloat16)

        for z in range(ROWS_PT // ZROWS):
            pltpu.sync_copy(zero_v,
                            acc.at[pl.ds(s * ROWS_PT + z * ZROWS, ZROWS)])
        plsc.subcore_barrier()

        def start_gather(kk, buf, sem):
            pltpu.async_copy(xw_hbm.at[src_v.at[kk]], buf, sem)

        def finish_chunk(kk, buf, sem, scaled, ssem, first_round):
            pltpu.make_async_copy(xw_hbm.at[src_v.at[kk]], buf, sem).wait()

            if first_round is None:
                pltpu.make_async_copy(scaled, acc.at[dst_v.at[kk]],
                                      ssem).wait()
            else:
                @pl.when(jnp.logical_not(first_round))
                def _drain():
                    pltpu.make_async_copy(scaled, acc.at[dst_v.at[kk]],
                                          ssem).wait()

            def scale_group(base):
                w16 = w_v[kk, pl.ds(base, L)]
                for e in range(L):
                    wsp = jnp.full((L,), w16[e], jnp.float32)
                    wb = plsc.pack(wsp, wsp,
                                   format=plsc.PackFormat.INTERLEAVED)
                    row = base + e
                    for q in range(D // (2 * L)):
                        sl = pl.ds(q * 2 * L, 2 * L)
                        scaled[row, sl] = buf[row, sl] * wb

            @pl.loop(0, CHUNK // L)
            def _scale(g):
                scale_group(g * L)

            if CHUNK % L:
                scale_group(CHUNK - L)

            pltpu.async_copy(scaled, acc.at[dst_v.at[kk]], ssem, add=True)

        start_gather(0, rows_a, sem_a)
        start_gather(1, rows_b, sem_b)
        start_gather(2, rows_c, sem_c)

        bufs = ((rows_a, sem_a), (rows_b, sem_b),
                (rows_c, sem_c), (rows_d, sem_d))

        sbufs = ((scaled_a, ssem_a), (scaled_b, ssem_b))

        @pl.loop(0, NCHUNK // 4)
        def _quad(q):
            k0 = 4 * q
            for j in range(4):
                nxt = k0 + j + 3
                buf, sem = bufs[(j + 3) % 4]
                if j == 0:
                    start_gather(nxt, buf, sem)
                else:
                    @pl.when(nxt < NCHUNK)
                    def _prefetch(nxt=nxt, buf=buf, sem=sem):
                        start_gather(nxt, buf, sem)
                scaled, ssem = sbufs[j % 2]
                first = (q == 0) if j < 2 else None
                finish_chunk(k0 + j, *bufs[j], scaled, ssem, first)

        pltpu.make_async_copy(scaled_a, acc.at[dst_v.at[0]], ssem_a).wait()
        pltpu.make_async_copy(scaled_b, acc.at[dst_v.at[0]], ssem_b).wait()

        plsc.subcore_barrier()
        pltpu.sync_copy(acc.at[pl.ds(s * ROWS_PT, ROWS_PT)],
                        out_hbm.at[c].at[pl.ds(s * ROWS_PT, ROWS_PT)])

    return k(xw, src2, dst2, w2)


def kernel(x, edge_index, edge_weight, W):
    rows = N_EDGES // CHUNK
    dst2 = edge_index[0].astype(jnp.int32).reshape(rows, CHUNK)
    src2 = edge_index[1].astype(jnp.int32).reshape(rows, CHUNK)
    w2 = edge_weight.reshape(rows, CHUNK)
    xw = _matmul_bf16(x, W)
    parts = _edge_agg(xw, src2, dst2, w2)
    return _combine(parts)
